# baseline (speedup 1.0000x reference)
import functools

import jax
import jax.numpy as jnp
from jax import lax
from jax.experimental import pallas as pl
from jax.experimental.pallas import tpu as pltpu
from jax.experimental.pallas import tpu_sc as plsc

F = 26
V = 100000
D = 16
B = 16384
FD = F * D
NCB = 4
PK = 512
H1, H2, H3 = 512, 256, 128
OUT = 128

NC = 2
NS = 16
NW = NC * NS
BPW = B // NW
SUB = 128
NSUB = BPW // SUB
WIN = 6


def _gather_body(nf, nslab, idx_hbm, tbl_hbm, out_hbm, idx_v, eidx_v, colr_v,
                 zero_v, sem):
    wid = lax.axis_index("s") * NC + lax.axis_index("c")
    pltpu.sync_copy(idx_hbm.at[:, pl.ds(wid * NSUB, NSUB), :], idx_v)

    if nf < nslab * 8:
        def z_body(i, _):
            zero_v[i // (BPW // D), pl.ds((i % (BPW // D)) * D, D)] = (
                jnp.zeros((D,), jnp.float32))
            return 0

        lax.fori_loop(0, D * (BPW // D), z_body, 0)
        for pf in range(nf, nslab * 8):
            pltpu.sync_copy(
                zero_v,
                out_hbm.at[nslab - 1, pl.ds((pf % 8) * D, D),
                           pl.ds(wid * BPW, BPW)],
            )

    def f_body(f, _):
        def build(d, _):
            base = (f * D + d) * V
            for c in range(NSUB):
                for kk in range(SUB // D):
                    sl = pl.ds(kk * D, D)
                    eidx_v[d * NSUB + c, sl] = idx_v[f, c, sl] + base
            return 0

        lax.fori_loop(0, D, build, 0)

        def s_body(i, _):
            cps = [
                pltpu.async_copy(
                    tbl_hbm.at[eidx_v.at[i * D + k]],
                    colr_v.at[i * NSUB + k // NSUB, pl.ds((k % NSUB) * SUB, SUB)],
                    sem,
                )
                for k in range(D)
            ]
            for cp in cps:
                cp.wait()
            return 0

        lax.fori_loop(0, NSUB, s_body, 0)
        pltpu.sync_copy(
            colr_v,
            out_hbm.at[f // 8, pl.ds((f % 8) * D, D), pl.ds(wid * BPW, BPW)],
        )
        return 0

    lax.fori_loop(0, nf, f_body, 0)


@functools.cache
def _get_gather(nf):
    nslab = (nf + 7) // 8

    @functools.partial(
        pl.kernel,
        out_type=jax.ShapeDtypeStruct((nslab, 128, B), jnp.float32),
        mesh=plsc.VectorSubcoreMesh(core_axis_name="c", subcore_axis_name="s"),
        scratch_types=[
            pltpu.VMEM((nf, NSUB, SUB), jnp.int32),
            pltpu.VMEM((D * NSUB, SUB), jnp.int32),
            pltpu.VMEM((D, BPW), jnp.float32),
            pltpu.VMEM((D, BPW), jnp.float32),
            pltpu.SemaphoreType.DMA,
        ],
        compiler_params=pltpu.CompilerParams(use_tc_tiling_on_sc=False),
    )
    def _gather(idx_hbm, tbl_hbm, out_hbm, idx_v, eidx_v, colr_v, zero_v, sem):
        _gather_body(nf, nslab, idx_hbm, tbl_hbm, out_hbm, idx_v, eidx_v,
                     colr_v, zero_v, sem)

    return _gather


BM = 1024


def _mlp_body(xa_ref, xb_ref, w1_ref, b1_ref, w2_ref, b2_ref, w3_ref, b3_ref,
              wt_ref, wb_ref, bo_ref, s_ref, o_ref):
    xt = jnp.concatenate(
        [xa_ref[0], xa_ref[1], xb_ref[0], xb_ref[1]], axis=0)
    h = jnp.maximum(jnp.dot(w1_ref[...], xt, preferred_element_type=jnp.float32)
                    + b1_ref[...], 0.0)
    h = jnp.maximum(jnp.dot(w2_ref[...], h, preferred_element_type=jnp.float32)
                    + b2_ref[...], 0.0)
    h = jnp.maximum(jnp.dot(w3_ref[...], h, preferred_element_type=jnp.float32)
                    + b3_ref[...], 0.0)
    st = jnp.dot(s_ref[...], xt, preferred_element_type=jnp.float32)
    qt = jnp.dot(s_ref[...], xt * xt, preferred_element_type=jnp.float32)
    fmt = (st * st - qt) * (0.5 / F)
    ot = (jnp.dot(wt_ref[...], h, preferred_element_type=jnp.float32)
          + jnp.dot(wb_ref[...], fmt, preferred_element_type=jnp.float32)
          + bo_ref[...])
    o_ref[...] = ot.T


def _mlp(xa, xb, W1t, b1, W2t, b2, W3t, b3, Wtt, Wbt, bo, St):
    full = lambda shape: pl.BlockSpec(shape, lambda i: (0,) * len(shape))
    return pl.pallas_call(
        _mlp_body,
        grid=(B // BM,),
        in_specs=[
            pl.BlockSpec((2, 128, BM), lambda i: (0, 0, i)),
            pl.BlockSpec((2, 128, BM), lambda i: (0, 0, i)),
            full((H1, PK)), full((H1, 1)),
            full((H2, H1)), full((H2, 1)),
            full((H3, H2)), full((H3, 1)),
            full((OUT, H3)), full((OUT, D)), full((OUT, 1)),
            full((D, PK)),
        ],
        out_specs=pl.BlockSpec((BM, OUT), lambda i: (i, 0)),
        out_shape=jax.ShapeDtypeStruct((B, OUT), jnp.float32),
    )(xa, xb, W1t, b1, W2t, b2, W3t, b3, Wtt, Wbt, bo, St)


FSPLIT = 16


def kernel(indices, tables, W1, b1, W2, b2, W3, b3, Wo, bo):
    idx3 = indices.astype(jnp.int32).T.reshape(F, B // SUB, SUB)
    tables_t = jnp.transpose(tables, (0, 2, 1))
    tbl_a = tables_t[:FSPLIT].reshape(FSPLIT * D * V)
    tbl_b = tables_t[FSPLIT:].reshape((F - FSPLIT) * D * V)
    xa = _get_gather(FSPLIT)(idx3[:FSPLIT], tbl_a)
    xb = _get_gather(F - FSPLIT)(idx3[FSPLIT:], tbl_b)
    pad = jnp.zeros((H1, PK - FD), jnp.float32)
    W1t = jnp.concatenate([W1.T, pad], axis=1)
    St = jnp.concatenate(
        [jnp.tile(jnp.eye(D, dtype=jnp.float32), (1, F)),
         jnp.zeros((D, PK - FD), jnp.float32)], axis=1)
    return _mlp(xa, xb, W1t, b1.reshape(H1, 1), W2.T, b2.reshape(H2, 1),
                W3.T, b3.reshape(H3, 1), Wo[:H3].T, Wo[H3:].T,
                bo.reshape(OUT, 1), St)


# speedup vs baseline: 1.0291x; 1.0291x over previous
import functools

import jax
import jax.numpy as jnp
from jax import lax
from jax.experimental import pallas as pl
from jax.experimental.pallas import tpu as pltpu
from jax.experimental.pallas import tpu_sc as plsc

F = 26
V = 100000
D = 16
B = 16384
FD = F * D
NCB = 4
PK = 512
H1, H2, H3 = 512, 256, 128
OUT = 128

NC = 2
NS = 16
NW = NC * NS
BPW = B // NW
SUB = 128
NSUB = BPW // SUB
WIN = 6


def _gather_body(nf, nslab, idx_hbm, tbl_hbm, out_hbm, idx_v, eidx_v, colr_v,
                 zero_v, sem):
    wid = lax.axis_index("s") * NC + lax.axis_index("c")
    pltpu.sync_copy(idx_hbm.at[:, pl.ds(wid * NSUB, NSUB), :], idx_v)

    if nf < nslab * 8:
        def z_body(i, _):
            zero_v[i // (BPW // D), pl.ds((i % (BPW // D)) * D, D)] = (
                jnp.zeros((D,), jnp.float32))
            return 0

        lax.fori_loop(0, D * (BPW // D), z_body, 0)
        for pf in range(nf, nslab * 8):
            pltpu.sync_copy(
                zero_v,
                out_hbm.at[nslab - 1, pl.ds((pf % 8) * D, D),
                           pl.ds(wid * BPW, BPW)],
            )

    def f_body(f, _):
        def build(d, _):
            base = (f * D + d) * V
            for c in range(NSUB):
                for kk in range(SUB // D):
                    sl = pl.ds(kk * D, D)
                    eidx_v[d * NSUB + c, sl] = idx_v[f, c, sl] + base
            return 0

        lax.fori_loop(0, D, build, 0)

        def s_body(i, _):
            for k in range(D):
                pltpu.async_copy(
                    tbl_hbm.at[eidx_v.at[i * D + k]],
                    colr_v.at[i * NSUB + k // NSUB, pl.ds((k % NSUB) * SUB, SUB)],
                    sem,
                )

            @pl.when(i >= 1)
            def _():
                pltpu.make_async_copy(
                    out_hbm.at[0, pl.ds(0, D), pl.ds(0, SUB)],
                    colr_v.at[:, pl.ds(0, SUB)],
                    sem,
                ).wait()

            return 0

        lax.fori_loop(0, NSUB, s_body, 0)
        pltpu.make_async_copy(
            out_hbm.at[0, pl.ds(0, D), pl.ds(0, SUB)],
            colr_v.at[:, pl.ds(0, SUB)],
            sem,
        ).wait()
        pltpu.sync_copy(
            colr_v,
            out_hbm.at[f // 8, pl.ds((f % 8) * D, D), pl.ds(wid * BPW, BPW)],
        )
        return 0

    lax.fori_loop(0, nf, f_body, 0)


@functools.cache
def _get_gather(nf):
    nslab = (nf + 7) // 8

    @functools.partial(
        pl.kernel,
        out_type=jax.ShapeDtypeStruct((nslab, 128, B), jnp.float32),
        mesh=plsc.VectorSubcoreMesh(core_axis_name="c", subcore_axis_name="s"),
        scratch_types=[
            pltpu.VMEM((nf, NSUB, SUB), jnp.int32),
            pltpu.VMEM((D * NSUB, SUB), jnp.int32),
            pltpu.VMEM((D, BPW), jnp.float32),
            pltpu.VMEM((D, BPW), jnp.float32),
            pltpu.SemaphoreType.DMA,
        ],
        compiler_params=pltpu.CompilerParams(use_tc_tiling_on_sc=False),
    )
    def _gather(idx_hbm, tbl_hbm, out_hbm, idx_v, eidx_v, colr_v, zero_v, sem):
        _gather_body(nf, nslab, idx_hbm, tbl_hbm, out_hbm, idx_v, eidx_v,
                     colr_v, zero_v, sem)

    return _gather


BM = 1024


def _mlp_body(xa_ref, xb_ref, w1_ref, b1_ref, w2_ref, b2_ref, w3_ref, b3_ref,
              wt_ref, wb_ref, bo_ref, s_ref, o_ref):
    xt = jnp.concatenate(
        [xa_ref[0], xa_ref[1], xb_ref[0], xb_ref[1]], axis=0)
    h = jnp.maximum(jnp.dot(w1_ref[...], xt, preferred_element_type=jnp.float32)
                    + b1_ref[...], 0.0)
    h = jnp.maximum(jnp.dot(w2_ref[...], h, preferred_element_type=jnp.float32)
                    + b2_ref[...], 0.0)
    h = jnp.maximum(jnp.dot(w3_ref[...], h, preferred_element_type=jnp.float32)
                    + b3_ref[...], 0.0)
    st = jnp.dot(s_ref[...], xt, preferred_element_type=jnp.float32)
    qt = jnp.dot(s_ref[...], xt * xt, preferred_element_type=jnp.float32)
    fmt = (st * st - qt) * (0.5 / F)
    ot = (jnp.dot(wt_ref[...], h, preferred_element_type=jnp.float32)
          + jnp.dot(wb_ref[...], fmt, preferred_element_type=jnp.float32)
          + bo_ref[...])
    o_ref[...] = ot.T


def _mlp(xa, xb, W1t, b1, W2t, b2, W3t, b3, Wtt, Wbt, bo, St):
    full = lambda shape: pl.BlockSpec(shape, lambda i: (0,) * len(shape))
    return pl.pallas_call(
        _mlp_body,
        grid=(B // BM,),
        in_specs=[
            pl.BlockSpec((2, 128, BM), lambda i: (0, 0, i)),
            pl.BlockSpec((2, 128, BM), lambda i: (0, 0, i)),
            full((H1, PK)), full((H1, 1)),
            full((H2, H1)), full((H2, 1)),
            full((H3, H2)), full((H3, 1)),
            full((OUT, H3)), full((OUT, D)), full((OUT, 1)),
            full((D, PK)),
        ],
        out_specs=pl.BlockSpec((BM, OUT), lambda i: (i, 0)),
        out_shape=jax.ShapeDtypeStruct((B, OUT), jnp.float32),
    )(xa, xb, W1t, b1, W2t, b2, W3t, b3, Wtt, Wbt, bo, St)


FSPLIT = 16


def kernel(indices, tables, W1, b1, W2, b2, W3, b3, Wo, bo):
    idx3 = indices.astype(jnp.int32).T.reshape(F, B // SUB, SUB)
    tables_t = jnp.transpose(tables, (0, 2, 1))
    tbl_a = tables_t[:FSPLIT].reshape(FSPLIT * D * V)
    tbl_b = tables_t[FSPLIT:].reshape((F - FSPLIT) * D * V)
    xa = _get_gather(FSPLIT)(idx3[:FSPLIT], tbl_a)
    xb = _get_gather(F - FSPLIT)(idx3[FSPLIT:], tbl_b)
    pad = jnp.zeros((H1, PK - FD), jnp.float32)
    W1t = jnp.concatenate([W1.T, pad], axis=1)
    St = jnp.concatenate(
        [jnp.tile(jnp.eye(D, dtype=jnp.float32), (1, F)),
         jnp.zeros((D, PK - FD), jnp.float32)], axis=1)
    return _mlp(xa, xb, W1t, b1.reshape(H1, 1), W2.T, b2.reshape(H2, 1),
                W3.T, b3.reshape(H3, 1), Wo[:H3].T, Wo[H3:].T,
                bo.reshape(OUT, 1), St)
